# Initial kernel scaffold; baseline (speedup 1.0000x reference)
import jax
import jax.numpy as jnp
from jax.experimental import pallas as pl


def kernel(x, edge_index, gcn_weight, gcn_bias, ln_weight):
    raise NotImplementedError("write your pallas kernel here")


# speedup vs baseline: 21.7331x; 21.7331x over previous
import functools

import jax
import jax.numpy as jnp
from jax import lax
from jax.experimental import pallas as pl
from jax.experimental.pallas import tpu as pltpu
from jax.experimental.pallas import tpu_sc as plsc

N = 10000
E = 320000
D = 128
DH = D // 2
NC, NS = 2, 16
NW = NC * NS
EPW = E // NW
EPT = E // NS
K = 80
NBA = EPW // K
NBC = EPT // K
NP = 10240
RPT = NP // NS
RB = 1000
GRID = N // RB

_mesh = plsc.VectorSubcoreMesh(core_axis_name="c", subcore_axis_name="s")
_sc_params = pltpu.CompilerParams(use_tc_tiling_on_sc=False)


def _zero_vec(ref, nwords):
    z = jnp.zeros((16,), jnp.float32)

    def body(i, carry):
        ref[pl.ds(i * 16, 16)] = z
        return carry

    lax.fori_loop(0, nwords // 16, body, 0)


def _zero_rows(ref, nrows, ncols):
    z = jnp.zeros((16,), jnp.float32)

    def body(i, carry):
        for jj in range(ncols // 16):
            ref[i, pl.ds(jj * 16, 16)] = z
        return carry

    lax.fori_loop(0, nrows, body, 0)


def _deg_body(dst_hbm, deg_hbm, idx_v, ones_v, buf_v, deg_sh, sem):
    c = lax.axis_index("c")
    s = lax.axis_index("s")
    wid = s * NC + c

    pltpu.async_copy(dst_hbm.at[wid], idx_v, sem)
    for jj in range(K // 16):
        ones_v[pl.ds(jj * 16, 16)] = jnp.full((16,), 1.0, jnp.float32)

    @pl.when(s == 0)
    def _():
        _zero_vec(buf_v, N)
        pltpu.sync_copy(buf_v, deg_sh)

    plsc.subcore_barrier()
    pltpu.make_async_copy(dst_hbm.at[wid], idx_v, sem).wait()

    def body(i, carry):
        pltpu.sync_copy(ones_v, deg_sh.at[idx_v.at[i]], add=True)
        return carry

    lax.fori_loop(0, NBA, body, 0)
    plsc.subcore_barrier()

    @pl.when(s == 0)
    def _():
        pltpu.sync_copy(deg_sh, buf_v)
        pltpu.sync_copy(buf_v, deg_hbm.at[pl.ds(c * N, N)])


_deg_call = functools.partial(
    pl.kernel,
    out_type=jax.ShapeDtypeStruct((NC * N,), jnp.float32),
    mesh=_mesh,
    compiler_params=_sc_params,
    scratch_types=[
        pltpu.VMEM((NBA, K), jnp.int32),
        pltpu.VMEM((K,), jnp.float32),
        pltpu.VMEM((N,), jnp.float32),
        pltpu.VMEM_SHARED((N,), jnp.float32),
        pltpu.SemaphoreType.DMA,
    ],
)(_deg_body)


def _msg_body(g_hbm, src_hbm, dst_hbm, acc_hbm,
              srcv, dstv, rows0, rows1, zb_v, acc_sh, sem0, sem1):
    c = lax.axis_index("c")
    s = lax.axis_index("s")

    pltpu.async_copy(src_hbm.at[s], srcv, sem0)
    pltpu.async_copy(dst_hbm.at[s], dstv, sem1)

    _zero_rows(zb_v, RPT // 5, DH)
    for r in range(5):
        pltpu.sync_copy(zb_v, acc_sh.at[pl.ds(s * RPT + r * (RPT // 5), RPT // 5)])

    pltpu.make_async_copy(src_hbm.at[s], srcv, sem0).wait()
    pltpu.make_async_copy(dst_hbm.at[s], dstv, sem1).wait()
    plsc.subcore_barrier()

    gh = g_hbm.at[c]
    pltpu.async_copy(gh.at[srcv.at[0]], rows0, sem0)

    def body(j, carry):
        i0 = 2 * j
        i1 = i0 + 1
        pltpu.make_async_copy(gh.at[srcv.at[i0]], rows0, sem0).wait()
        pltpu.async_copy(gh.at[srcv.at[i1]], rows1, sem1)
        pltpu.sync_copy(rows0, acc_sh.at[dstv.at[i0]], add=True)
        pltpu.make_async_copy(gh.at[srcv.at[i1]], rows1, sem1).wait()

        @pl.when(i1 + 1 < NBC)
        def _():
            pltpu.async_copy(gh.at[srcv.at[i1 + 1]], rows0, sem0)

        pltpu.sync_copy(rows1, acc_sh.at[dstv.at[i1]], add=True)
        return carry

    lax.fori_loop(0, NBC // 2, body, 0)

    plsc.subcore_barrier()
    pltpu.sync_copy(acc_sh.at[pl.ds(s * RPT, RPT)],
                    acc_hbm.at[c, pl.ds(s * RPT, RPT)])


_msg_call = functools.partial(
    pl.kernel,
    out_type=jax.ShapeDtypeStruct((NC, NP, DH), jnp.float32),
    mesh=_mesh,
    compiler_params=_sc_params,
    scratch_types=[
        pltpu.VMEM((NBC, K), jnp.int32),
        pltpu.VMEM((NBC, K), jnp.int32),
        pltpu.VMEM((K, DH), jnp.float32),
        pltpu.VMEM((K, DH), jnp.float32),
        pltpu.VMEM((RPT // 5, DH), jnp.float32),
        pltpu.VMEM_SHARED((NP, DH), jnp.float32),
        pltpu.SemaphoreType.DMA,
        pltpu.SemaphoreType.DMA,
    ],
)(_msg_body)


def _g_body(x_ref, w_ref, deg_ref, g_ref):
    dp = deg_ref[...]
    dis = lax.rsqrt(dp[0] + dp[1] + 1.0)
    h = jnp.dot(x_ref[...], w_ref[...], preferred_element_type=jnp.float32)
    h = h * dis
    g_ref[0, :, :] = h[:, :DH]
    g_ref[1, :, :] = h[:, DH:]


def _g_call(x, w, deg3):
    return pl.pallas_call(
        _g_body,
        grid=(GRID,),
        in_specs=[
            pl.BlockSpec((RB, D), lambda i: (i, 0)),
            pl.BlockSpec((D, D), lambda i: (0, 0)),
            pl.BlockSpec((NC, RB, 1), lambda i: (0, i, 0)),
        ],
        out_specs=pl.BlockSpec((NC, RB, DH), lambda i: (0, i, 0)),
        out_shape=jax.ShapeDtypeStruct((NC, N, DH), jnp.float32),
    )(x, w, deg3)


def _out_body(acc_ref, g_ref, x_ref, deg_ref, wt_ref, b_ref, o_ref):
    dp = deg_ref[...]
    dis = lax.rsqrt(dp[0] + dp[1] + 1.0)
    t = (acc_ref[...] + g_ref[...]) * dis[None] + b_ref[...]
    r = jnp.maximum(t, 0.0)
    r2 = jnp.concatenate([r[0], r[1]], axis=1)
    y = jnp.dot(r2, wt_ref[...], preferred_element_type=jnp.float32) + x_ref[...]
    o_ref[...] = jnp.maximum(y, 0.0)


def _out_call(acc, g, x, deg3, ln_wt, bias3):
    return pl.pallas_call(
        _out_body,
        grid=(GRID,),
        in_specs=[
            pl.BlockSpec((NC, RB, DH), lambda i: (0, i, 0)),
            pl.BlockSpec((NC, RB, DH), lambda i: (0, i, 0)),
            pl.BlockSpec((RB, D), lambda i: (i, 0)),
            pl.BlockSpec((NC, RB, 1), lambda i: (0, i, 0)),
            pl.BlockSpec((D, D), lambda i: (0, 0)),
            pl.BlockSpec((NC, 1, DH), lambda i: (0, 0, 0)),
        ],
        out_specs=pl.BlockSpec((RB, D), lambda i: (i, 0)),
        out_shape=jax.ShapeDtypeStruct((N, D), jnp.float32),
    )(acc, g, x, deg3, ln_wt, bias3)


def kernel(x, edge_index, gcn_weight, gcn_bias, ln_weight):
    src = edge_index[0].astype(jnp.int32)
    dst = edge_index[1].astype(jnp.int32)
    dst_a = dst.reshape(NW, NBA, K)
    src_c = src.reshape(NS, NBC, K)
    dst_c = dst.reshape(NS, NBC, K)
    deg = _deg_call(dst_a)
    deg3 = deg.reshape(NC, N, 1)
    g = _g_call(x, gcn_weight, deg3)
    acc = _msg_call(g, src_c, dst_c)
    return _out_call(acc, g, x, deg3,
                     ln_weight.T, gcn_bias.reshape(NC, 1, DH))


# speedup vs baseline: 30.6233x; 1.4091x over previous
import functools

import jax
import jax.numpy as jnp
from jax import lax
from jax.experimental import pallas as pl
from jax.experimental.pallas import tpu as pltpu
from jax.experimental.pallas import tpu_sc as plsc

N = 10000
E = 320000
D = 128
DH = D // 2
NC, NS = 2, 16
NW = NC * NS
EPW = E // NW
EPT = E // NS
K = 80
NBA = EPW // K
NBC = EPT // K
NP = 10240
RPT = NP // NS
RB = 1000
GRID = N // RB

_mesh = plsc.VectorSubcoreMesh(core_axis_name="c", subcore_axis_name="s")
_sc_params = pltpu.CompilerParams(use_tc_tiling_on_sc=False)


def _zero_vec(ref, nwords):
    z = jnp.zeros((16,), jnp.float32)

    def body(i, carry):
        ref[pl.ds(i * 16, 16)] = z
        return carry

    lax.fori_loop(0, nwords // 16, body, 0)


def _zero_rows(ref, nrows, ncols):
    z = jnp.zeros((16,), jnp.float32)

    def body(i, carry):
        for jj in range(ncols // 16):
            ref[i, pl.ds(jj * 16, 16)] = z
        return carry

    lax.fori_loop(0, nrows, body, 0)


def _deg_body(dst_hbm, deg_hbm, idx_v, ones_v, buf_v, deg_sh, sem):
    c = lax.axis_index("c")
    s = lax.axis_index("s")
    wid = s * NC + c

    pltpu.async_copy(dst_hbm.at[wid], idx_v, sem)
    for jj in range(K // 16):
        ones_v[pl.ds(jj * 16, 16)] = jnp.full((16,), 1.0, jnp.float32)

    @pl.when(s == 0)
    def _():
        _zero_vec(buf_v, N)
        pltpu.sync_copy(buf_v, deg_sh)

    plsc.subcore_barrier()
    pltpu.make_async_copy(dst_hbm.at[wid], idx_v, sem).wait()

    def body(i, carry):
        @pl.when(i < NBA)
        def _():
            pltpu.async_copy(ones_v, deg_sh.at[idx_v.at[i]], sem, add=True)

        @pl.when(i >= 8)
        def _():
            pltpu.make_async_copy(ones_v, deg_sh.at[idx_v.at[0]], sem).wait()

        return carry

    lax.fori_loop(0, NBA + 8, body, 0)
    plsc.subcore_barrier()

    @pl.when(s == 0)
    def _():
        pltpu.sync_copy(deg_sh, buf_v)
        pltpu.sync_copy(buf_v, deg_hbm.at[pl.ds(c * N, N)])


_deg_call = functools.partial(
    pl.kernel,
    out_type=jax.ShapeDtypeStruct((NC * N,), jnp.float32),
    mesh=_mesh,
    compiler_params=_sc_params,
    scratch_types=[
        pltpu.VMEM((NBA, K), jnp.int32),
        pltpu.VMEM((K,), jnp.float32),
        pltpu.VMEM((N,), jnp.float32),
        pltpu.VMEM_SHARED((N,), jnp.float32),
        pltpu.SemaphoreType.DMA,
    ],
)(_deg_body)


def _msg_body(g_hbm, src_hbm, dst_hbm, acc_hbm,
              srcv, dstv, r0, r1, r2, r3, zb_v, acc_sh,
              g0, g1, g2, g3, s0, s1, s2, s3):
    c = lax.axis_index("c")
    s = lax.axis_index("s")
    rows = [r0, r1, r2, r3]
    gsem = [g0, g1, g2, g3]
    ssem = [s0, s1, s2, s3]

    pltpu.async_copy(src_hbm.at[s], srcv, g0)
    pltpu.async_copy(dst_hbm.at[s], dstv, g1)

    _zero_rows(zb_v, RPT // 5, DH)
    for r in range(5):
        pltpu.sync_copy(zb_v, acc_sh.at[pl.ds(s * RPT + r * (RPT // 5), RPT // 5)])

    pltpu.make_async_copy(src_hbm.at[s], srcv, g0).wait()
    pltpu.make_async_copy(dst_hbm.at[s], dstv, g1).wait()
    plsc.subcore_barrier()

    gh = g_hbm.at[c]

    def start_gather(i, b):
        pltpu.async_copy(gh.at[srcv.at[i]], rows[b], gsem[b])

    def wait_gather(i, b):
        pltpu.make_async_copy(gh.at[srcv.at[i]], rows[b], gsem[b]).wait()

    def start_scat(i, b):
        pltpu.async_copy(rows[b], acc_sh.at[dstv.at[i]], ssem[b], add=True)

    def wait_scat(i, b):
        pltpu.make_async_copy(rows[b], acc_sh.at[dstv.at[i]], ssem[b]).wait()

    start_gather(0, 0)
    start_gather(1, 1)
    wait_gather(0, 0)
    start_scat(0, 0)
    start_gather(2, 2)
    wait_gather(1, 1)
    start_scat(1, 1)
    start_gather(3, 3)

    def body(q, carry):
        i_base = 2 + 4 * q
        for r in range(4):
            i = i_base + r
            b = (2 + r) % 4
            wait_gather(i, b)
            start_scat(i, b)
            wait_scat(i - 2, r)
            start_gather(i + 2, r)
        return carry

    lax.fori_loop(0, (NBC - 6) // 4, body, 0)

    for r in range(4):
        i = NBC - 4 + r
        b = (2 + r) % 4
        wait_gather(i, b)
        start_scat(i, b)
        wait_scat(i - 2, r)
        if r < 2:
            start_gather(i + 2, r)
    wait_scat(NBC - 2, 0)
    wait_scat(NBC - 1, 1)

    plsc.subcore_barrier()
    pltpu.sync_copy(acc_sh.at[pl.ds(s * RPT, RPT)],
                    acc_hbm.at[c, pl.ds(s * RPT, RPT)])


_msg_call = functools.partial(
    pl.kernel,
    out_type=jax.ShapeDtypeStruct((NC, NP, DH), jnp.float32),
    mesh=_mesh,
    compiler_params=_sc_params,
    scratch_types=[
        pltpu.VMEM((NBC, K), jnp.int32),
        pltpu.VMEM((NBC, K), jnp.int32),
        pltpu.VMEM((K, DH), jnp.float32),
        pltpu.VMEM((K, DH), jnp.float32),
        pltpu.VMEM((K, DH), jnp.float32),
        pltpu.VMEM((K, DH), jnp.float32),
        pltpu.VMEM((RPT // 5, DH), jnp.float32),
        pltpu.VMEM_SHARED((NP, DH), jnp.float32),
        pltpu.SemaphoreType.DMA,
        pltpu.SemaphoreType.DMA,
        pltpu.SemaphoreType.DMA,
        pltpu.SemaphoreType.DMA,
        pltpu.SemaphoreType.DMA,
        pltpu.SemaphoreType.DMA,
        pltpu.SemaphoreType.DMA,
        pltpu.SemaphoreType.DMA,
    ],
)(_msg_body)


def _g_body(x_ref, w_ref, deg_ref, g_ref):
    dp = deg_ref[...]
    dis = lax.rsqrt(dp[0] + dp[1] + 1.0)
    h = jnp.dot(x_ref[...], w_ref[...], preferred_element_type=jnp.float32)
    h = h * dis
    g_ref[0, :, :] = h[:, :DH]
    g_ref[1, :, :] = h[:, DH:]


def _g_call(x, w, deg3):
    return pl.pallas_call(
        _g_body,
        grid=(GRID,),
        in_specs=[
            pl.BlockSpec((RB, D), lambda i: (i, 0)),
            pl.BlockSpec((D, D), lambda i: (0, 0)),
            pl.BlockSpec((NC, RB, 1), lambda i: (0, i, 0)),
        ],
        out_specs=pl.BlockSpec((NC, RB, DH), lambda i: (0, i, 0)),
        out_shape=jax.ShapeDtypeStruct((NC, N, DH), jnp.float32),
    )(x, w, deg3)


def _out_body(acc_ref, g_ref, x_ref, deg_ref, wt_ref, b_ref, o_ref):
    dp = deg_ref[...]
    dis = lax.rsqrt(dp[0] + dp[1] + 1.0)
    t = (acc_ref[...] + g_ref[...]) * dis[None] + b_ref[...]
    r = jnp.maximum(t, 0.0)
    r2 = jnp.concatenate([r[0], r[1]], axis=1)
    y = jnp.dot(r2, wt_ref[...], preferred_element_type=jnp.float32) + x_ref[...]
    o_ref[...] = jnp.maximum(y, 0.0)


def _out_call(acc, g, x, deg3, ln_wt, bias3):
    return pl.pallas_call(
        _out_body,
        grid=(GRID,),
        in_specs=[
            pl.BlockSpec((NC, RB, DH), lambda i: (0, i, 0)),
            pl.BlockSpec((NC, RB, DH), lambda i: (0, i, 0)),
            pl.BlockSpec((RB, D), lambda i: (i, 0)),
            pl.BlockSpec((NC, RB, 1), lambda i: (0, i, 0)),
            pl.BlockSpec((D, D), lambda i: (0, 0)),
            pl.BlockSpec((NC, 1, DH), lambda i: (0, 0, 0)),
        ],
        out_specs=pl.BlockSpec((RB, D), lambda i: (i, 0)),
        out_shape=jax.ShapeDtypeStruct((N, D), jnp.float32),
    )(acc, g, x, deg3, ln_wt, bias3)


def kernel(x, edge_index, gcn_weight, gcn_bias, ln_weight):
    src = edge_index[0].astype(jnp.int32)
    dst = edge_index[1].astype(jnp.int32)
    dst_a = dst.reshape(NW, NBA, K)
    src_c = src.reshape(NS, NBC, K)
    dst_c = dst.reshape(NS, NBC, K)
    deg = _deg_call(dst_a)
    deg3 = deg.reshape(NC, N, 1)
    g = _g_call(x, gcn_weight, deg3)
    acc = _msg_call(g, src_c, dst_c)
    return _out_call(acc, g, x, deg3,
                     ln_weight.T, gcn_bias.reshape(NC, 1, DH))
